# baseline (speedup 1.0000x reference)
import jax
import jax.numpy as jnp
from jax.experimental import pallas as pl
from jax.experimental.pallas import tpu as pltpu


def _cheb_kernel(adj_ref, data_ref, w_ref, b_ref, out_ref,
                 lt_ref, x0c_ref, x1c_ref):
    i = pl.program_id(0)
    nb = pl.num_programs(0) - 1

    @pl.when(i == 0)
    def _prep():
        adj = adj_ref[...]
        deg = jnp.sum(adj, axis=1, keepdims=True)
        dinv = jnp.where(deg > 0, deg ** -0.5, 0.0)
        s = dinv * adj
        lt_ref[...] = (-dinv) * s.T

    @pl.when(i > 0)
    def _finish_prev():
        x0 = x0c_ref[...]
        x1 = x1c_ref[...]
        x2 = 2.0 * jnp.dot(lt_ref[...], x1,
                           preferred_element_type=jnp.float32) - x0
        z = jnp.concatenate([x0, x1, x2], axis=1)
        out_ref[0] = jnp.dot(z, w_ref[...],
                             preferred_element_type=jnp.float32) + b_ref[...]

    @pl.when(i < nb)
    def _start_next():
        x0n = data_ref[0]
        x1c_ref[...] = jnp.dot(lt_ref[...], x0n,
                               preferred_element_type=jnp.float32)
        x0c_ref[...] = x0n


def kernel(data, adj, W, b):
    B, N, F_IN = data.shape
    K, _, F_OUT = W.shape

    return pl.pallas_call(
        _cheb_kernel,
        grid=(B + 1,),
        in_specs=[
            pl.BlockSpec((N, N), lambda i: (0, 0)),
            pl.BlockSpec((1, N, F_IN),
                         lambda i: (jnp.minimum(i, B - 1), 0, 0)),
            pl.BlockSpec((K * F_IN, F_OUT), lambda i: (0, 0)),
            pl.BlockSpec((1, F_OUT), lambda i: (0, 0)),
        ],
        out_specs=pl.BlockSpec((1, N, F_OUT),
                               lambda i: (jnp.maximum(i - 1, 0), 0, 0)),
        out_shape=jax.ShapeDtypeStruct((B, N, F_OUT), jnp.float32),
        scratch_shapes=[pltpu.VMEM((N, N), jnp.float32),
                        pltpu.VMEM((N, F_IN), jnp.float32),
                        pltpu.VMEM((N, F_IN), jnp.float32)],
        compiler_params=pltpu.CompilerParams(
            dimension_semantics=("arbitrary",),
        ),
    )(adj, data, W.reshape(K * F_IN, F_OUT), b.reshape(1, F_OUT))


# speedup vs baseline: 1.0956x; 1.0956x over previous
import jax
import jax.numpy as jnp
from jax.experimental import pallas as pl
from jax.experimental.pallas import tpu as pltpu


def _cheb_kernel(adj_ref, data_ref, w_ref, b_ref, out_ref,
                 lt_ref, x0c_ref, x1c_ref):
    i = pl.program_id(0)
    nb = pl.num_programs(0) - 1

    @pl.when(i == 0)
    def _prep():
        adj = adj_ref[...]
        deg = jnp.sum(adj, axis=1, keepdims=True)
        dinv = jnp.where(deg > 0, deg ** -0.5, 0.0)
        s = dinv * adj
        lt_ref[...] = (-dinv) * s.T

    del i, nb
    x0 = x0c_ref[...]
    x1 = x1c_ref[...]
    x0n = data_ref[0]
    x2 = 2.0 * jnp.dot(lt_ref[...], x1,
                       preferred_element_type=jnp.float32) - x0
    x1n = jnp.dot(lt_ref[...], x0n,
                  preferred_element_type=jnp.float32)
    z = jnp.concatenate([x0, x1, x2], axis=1)
    out_ref[0] = jnp.dot(z, w_ref[...],
                         preferred_element_type=jnp.float32) + b_ref[...]
    x1c_ref[...] = x1n
    x0c_ref[...] = x0n


def kernel(data, adj, W, b):
    B, N, F_IN = data.shape
    K, _, F_OUT = W.shape

    return pl.pallas_call(
        _cheb_kernel,
        grid=(B + 1,),
        in_specs=[
            pl.BlockSpec((N, N), lambda i: (0, 0)),
            pl.BlockSpec((1, N, F_IN),
                         lambda i: (jnp.minimum(i, B - 1), 0, 0)),
            pl.BlockSpec((K * F_IN, F_OUT), lambda i: (0, 0)),
            pl.BlockSpec((1, F_OUT), lambda i: (0, 0)),
        ],
        out_specs=pl.BlockSpec((1, N, F_OUT),
                               lambda i: (jnp.maximum(i - 1, 0), 0, 0)),
        out_shape=jax.ShapeDtypeStruct((B, N, F_OUT), jnp.float32),
        scratch_shapes=[pltpu.VMEM((N, N), jnp.float32),
                        pltpu.VMEM((N, F_IN), jnp.float32),
                        pltpu.VMEM((N, F_IN), jnp.float32)],
        compiler_params=pltpu.CompilerParams(
            dimension_semantics=("arbitrary",),
        ),
    )(adj, data, W.reshape(K * F_IN, F_OUT), b.reshape(1, F_OUT))


# speedup vs baseline: 1.1011x; 1.0051x over previous
import jax
import jax.numpy as jnp
from jax.experimental import pallas as pl
from jax.experimental.pallas import tpu as pltpu


def _cheb_kernel(adj_ref, data_ref, w_ref, b_ref, out_ref,
                 lt_ref, x0c_ref, x1c_ref):
    i = pl.program_id(0)
    nb = pl.num_programs(0) - 1

    @pl.when(i == 0)
    def _prep():
        adj = adj_ref[...]
        deg = jnp.sum(adj, axis=1, keepdims=True)
        dinv = jnp.where(deg > 0, deg ** -0.5, 0.0)
        s = dinv * adj
        lt_ref[...] = (-dinv) * s.T

    del i, nb
    x0 = x0c_ref[...]
    x1 = x1c_ref[...]
    x0n = data_ref[0]
    x2 = 2.0 * jnp.dot(lt_ref[...], x1,
                       preferred_element_type=jnp.float32) - x0
    x1n = jnp.dot(lt_ref[...], x0n,
                  preferred_element_type=jnp.float32)
    acc = jnp.dot(x0, w_ref[0], preferred_element_type=jnp.float32)
    acc = acc + jnp.dot(x1, w_ref[1], preferred_element_type=jnp.float32)
    acc = acc + jnp.dot(x2, w_ref[2], preferred_element_type=jnp.float32)
    out_ref[0] = acc + b_ref[...]
    x1c_ref[...] = x1n
    x0c_ref[...] = x0n


def kernel(data, adj, W, b):
    B, N, F_IN = data.shape
    K, _, F_OUT = W.shape

    return pl.pallas_call(
        _cheb_kernel,
        grid=(B + 1,),
        in_specs=[
            pl.BlockSpec((N, N), lambda i: (0, 0)),
            pl.BlockSpec((1, N, F_IN),
                         lambda i: (jnp.minimum(i, B - 1), 0, 0)),
            pl.BlockSpec((K, F_IN, F_OUT), lambda i: (0, 0, 0)),
            pl.BlockSpec((1, F_OUT), lambda i: (0, 0)),
        ],
        out_specs=pl.BlockSpec((1, N, F_OUT),
                               lambda i: (jnp.maximum(i - 1, 0), 0, 0)),
        out_shape=jax.ShapeDtypeStruct((B, N, F_OUT), jnp.float32),
        scratch_shapes=[pltpu.VMEM((N, N), jnp.float32),
                        pltpu.VMEM((N, F_IN), jnp.float32),
                        pltpu.VMEM((N, F_IN), jnp.float32)],
        compiler_params=pltpu.CompilerParams(
            dimension_semantics=("arbitrary",),
        ),
    )(adj, data, W, b.reshape(1, F_OUT))
